# baseline (speedup 1.0000x reference)
import functools

import jax
import jax.numpy as jnp
from jax import lax
from jax.experimental import pallas as pl
from jax.experimental.pallas import tpu as pltpu
from jax.experimental.pallas import tpu_sc as plsc

N = 10000
D = 128
E = 320000
NC = 2
NS = 16
L = 16
NW = NC * NS
CH = 128
NCHUNK = 84
EPW = CH * NCHUNK
NP = 10112
RPT = NP // NS
EPT = E // NW

_mesh = plsc.VectorSubcoreMesh(core_axis_name="c", subcore_axis_name="s")


@functools.partial(
    pl.kernel,
    out_type=jax.ShapeDtypeStruct((NW, NP), jnp.float32),
    mesh=_mesh,
    scratch_types=[
        pltpu.VMEM((EPT,), jnp.int32),
        pltpu.VMEM((NP,), jnp.float32),
    ],
    compiler_params=pltpu.CompilerParams(needs_layout_passes=False),
)
def _hist(col_hbm, out_hbm, col_v, hist):
    cid = lax.axis_index("c")
    sid = lax.axis_index("s")
    wid = cid * NS + sid
    pltpu.sync_copy(col_hbm.at[pl.ds(wid * EPT, EPT)], col_v)

    zero16 = jnp.zeros((L,), jnp.float32)

    def zb(i, carry):
        hist[pl.ds(i * L, L)] = zero16
        return carry

    lax.fori_loop(0, NP // L, zb, 0)

    ones16 = jnp.ones((L,), jnp.float32)

    def body(i, carry):
        idx = col_v[pl.ds(i * L, L)]
        plsc.addupdate_scatter(hist, [idx], ones16)
        return carry

    lax.fori_loop(0, EPT // L, body, 0)
    pltpu.sync_copy(hist, out_hbm.at[wid])


@functools.partial(
    pl.kernel,
    out_type=jax.ShapeDtypeStruct((NC, NP, D), jnp.float32),
    mesh=_mesh,
    scratch_types=[
        pltpu.VMEM((CH,), jnp.int32),
        pltpu.VMEM((CH,), jnp.int32),
        pltpu.VMEM((CH,), jnp.int32),
        pltpu.VMEM((CH,), jnp.int32),
        pltpu.VMEM((CH,), jnp.int32),
        pltpu.VMEM((CH,), jnp.int32),
        pltpu.VMEM((CH, D), jnp.float32),
        pltpu.VMEM((CH, D), jnp.float32),
        pltpu.VMEM((CH, D), jnp.float32),
        pltpu.VMEM_SHARED((NP, D), jnp.float32),
        pltpu.SemaphoreType.DMA,
        pltpu.SemaphoreType.DMA,
        pltpu.SemaphoreType.DMA,
        pltpu.SemaphoreType.DMA,
    ],
    compiler_params=pltpu.CompilerParams(needs_layout_passes=False),
)
def _scatter(row_hbm, col_hbm, g_hbm, zeros_hbm, out_hbm,
             rb0, cb0, rb1, cb1, rb2, cb2, buf0, buf1, buf2, acc,
             semg0, semg1, semg2, semi):
    cid = lax.axis_index("c")
    sid = lax.axis_index("s")
    wid = cid * NS + sid
    base = sid * RPT
    pltpu.sync_copy(zeros_hbm.at[pl.ds(base, RPT)], acc.at[pl.ds(base, RPT)])
    plsc.subcore_barrier()

    rbs = (rb0, rb1, rb2)
    cbs = (cb0, cb1, cb2)
    bufs = (buf0, buf1, buf2)
    semg = (semg0, semg1, semg2)

    def body(j, carry):
        off = (wid * NCHUNK + 6 * j) * CH
        ii = []
        for k in (0, 1, 2):
            ii.append(pltpu.async_copy(
                row_hbm.at[pl.ds(off + k * CH, CH)], rbs[k], semi))
            ii.append(pltpu.async_copy(
                col_hbm.at[pl.ds(off + k * CH, CH)], cbs[k], semi))
        for i in ii:
            i.wait()
        dd = [pltpu.async_copy(g_hbm.at[rbs[k]], bufs[k], semg[k])
              for k in (0, 1, 2)]
        jj = []
        dd2 = []
        for k in (0, 1, 2):
            dd[k].wait()
            pltpu.sync_copy(bufs[k], acc.at[cbs[k]], add=True)
            o2 = off + (3 + k) * CH
            jA = pltpu.async_copy(row_hbm.at[pl.ds(o2, CH)], rbs[k], semi)
            jB = pltpu.async_copy(col_hbm.at[pl.ds(o2, CH)], cbs[k], semi)
            jA.wait()
            jB.wait()
            dd2.append(pltpu.async_copy(g_hbm.at[rbs[k]], bufs[k], semg[k]))
        for k in (0, 1, 2):
            dd2[k].wait()
            pltpu.sync_copy(bufs[k], acc.at[cbs[k]], add=True)
        return carry

    lax.fori_loop(0, NCHUNK // 6, body, 0)
    plsc.subcore_barrier()
    pltpu.sync_copy(acc.at[pl.ds(base, RPT)],
                    out_hbm.at[cid].at[pl.ds(base, RPT)])


def _prep_body(hist_ref, emb_ref, w_ref, dis_ref, g_ref):
    deg = 1.0 + jnp.sum(hist_ref[...], axis=0)
    dis = lax.rsqrt(deg)[:N, None]
    dis_ref[...] = dis
    h = jnp.dot(emb_ref[...], w_ref[...], preferred_element_type=jnp.float32)
    g_ref[...] = h * dis


def _mid_body(acc_ref, g_ref, dis_ref, b_ref, lnw_ref, lnb_ref, wn_ref,
              gn_ref):
    dis = dis_ref[...]
    s = acc_ref[0, :N, :] + acc_ref[1, :N, :] + g_ref[...]
    y = dis * s + b_ref[...]
    m = jnp.mean(y)
    v = jnp.mean((y - m) ** 2)
    z = (y - m) * lax.rsqrt(v + 1e-5) * lnw_ref[...] + lnb_ref[...]
    z = jnp.maximum(z, 0.0)
    gn_ref[...] = jnp.dot(z, wn_ref[...],
                          preferred_element_type=jnp.float32) * dis


def _fin_body(acc_ref, g_ref, dis_ref, b_ref, lnw_ref, lnb_ref,
              tw1_ref, tb1_ref, tw2_ref, tb2_ref, emb_ref, out_ref):
    s = acc_ref[0, :N, :] + acc_ref[1, :N, :] + g_ref[...]
    y = dis_ref[...] * s + b_ref[...]
    m = jnp.mean(y)
    v = jnp.mean((y - m) ** 2)
    z = (y - m) * lax.rsqrt(v + 1e-5) * lnw_ref[...] + lnb_ref[...]
    t = jnp.dot(z, tw1_ref[...], preferred_element_type=jnp.float32)
    t = jnp.maximum(t + tb1_ref[...], 0.0)
    out_ref[...] = (jnp.dot(t, tw2_ref[...], preferred_element_type=jnp.float32)
                    + tb2_ref[...] + emb_ref[...])


_f32 = jnp.float32
_prep = pl.pallas_call(
    _prep_body,
    out_shape=(jax.ShapeDtypeStruct((N, 1), _f32),
               jax.ShapeDtypeStruct((N, D), _f32)))
_mid = pl.pallas_call(
    _mid_body, out_shape=jax.ShapeDtypeStruct((N, D), _f32))
_fin = pl.pallas_call(
    _fin_body, out_shape=jax.ShapeDtypeStruct((N, D), _f32))


def kernel(edge_index, emb, W0, b0, W1, b1, W2, b2, lnw0, lnb0, lnw1, lnb1,
           lnw2, lnb2, tW1, tb1, tW2, tb2):
    row = edge_index[0]
    col = edge_index[1]
    pad = EPW - EPT
    rowf = jnp.concatenate(
        [row.reshape(NW, EPT), jnp.zeros((NW, pad), jnp.int32)],
        axis=1).reshape(-1)
    colf = jnp.concatenate(
        [col.reshape(NW, EPT), jnp.full((NW, pad), N, jnp.int32)],
        axis=1).reshape(-1)
    zeros = jnp.zeros((NP, D), _f32)

    hist = _hist(col)
    dis, g = _prep(hist, emb, W0)
    layers = [(b0, lnw0, lnb0), (b1, lnw1, lnb1), (b2, lnw2, lnb2)]
    nxt = [W1, W2]
    for i in range(3):
        acc = _scatter(rowf, colf, g, zeros)
        b, lw, lb = layers[i]
        if i < 2:
            g = _mid(acc, g, dis, b, lw, lb, nxt[i])
        else:
            return _fin(acc, g, dis, b, lw, lb, tW1, tb1, tW2, tb2, emb)


# speedup vs baseline: 2.3116x; 2.3116x over previous
import functools

import jax
import jax.numpy as jnp
from jax import lax
from jax.experimental import pallas as pl
from jax.experimental.pallas import tpu as pltpu
from jax.experimental.pallas import tpu_sc as plsc

N = 10000
D = 128
E = 320000
NC = 2
NS = 16
L = 16
NW = NC * NS
CH = 128
NCHUNK = 80
EPW = CH * NCHUNK
NP = 10112
RPT = NP // NS
EPT = E // NW
SHIFT = 14
MASK = (1 << SHIFT) - 1

_mesh = plsc.VectorSubcoreMesh(core_axis_name="c", subcore_axis_name="s")


@functools.partial(
    pl.kernel,
    out_type=jax.ShapeDtypeStruct((NW, NP), jnp.float32),
    mesh=_mesh,
    scratch_types=[
        pltpu.VMEM((EPT,), jnp.int32),
        pltpu.VMEM((NP,), jnp.float32),
    ],
    compiler_params=pltpu.CompilerParams(needs_layout_passes=False),
)
def _hist(col_hbm, out_hbm, col_v, hist):
    cid = lax.axis_index("c")
    sid = lax.axis_index("s")
    wid = cid * NS + sid
    pltpu.sync_copy(col_hbm.at[pl.ds(wid * EPT, EPT)], col_v)

    zero16 = jnp.zeros((L,), jnp.float32)

    def zb(i, carry):
        hist[pl.ds(i * L, L)] = zero16
        return carry

    lax.fori_loop(0, NP // L, zb, 0)

    ones16 = jnp.ones((L,), jnp.float32)

    def body(i, carry):
        idx = col_v[pl.ds(i * L, L)]
        plsc.addupdate_scatter(hist, [idx], ones16)
        return carry

    lax.fori_loop(0, EPT // L, body, 0)
    pltpu.sync_copy(hist, out_hbm.at[wid])


@functools.partial(
    pl.kernel,
    out_type=jax.ShapeDtypeStruct((NC, NP, D), jnp.float32),
    mesh=_mesh,
    scratch_types=[
        pltpu.VMEM((NCHUNK, CH), jnp.int32),
        pltpu.VMEM((2, CH), jnp.int32),
        pltpu.VMEM((2, CH), jnp.int32),
        pltpu.VMEM((CH, D), jnp.float32),
        pltpu.VMEM((CH, D), jnp.float32),
        pltpu.VMEM_SHARED((NP, D), jnp.float32),
        pltpu.SemaphoreType.DMA,
        pltpu.SemaphoreType.DMA,
        pltpu.SemaphoreType.DMA,
        pltpu.SemaphoreType.DMA,
    ],
    compiler_params=pltpu.CompilerParams(needs_layout_passes=False),
)
def _scatter(pk_hbm, g_hbm, zeros_hbm, out_hbm,
             pk_v, rbuf, cbuf, buf0, buf1, acc, semg0, semg1, sems0, sems1):
    cid = lax.axis_index("c")
    sid = lax.axis_index("s")
    wid = cid * NS + sid
    base = sid * RPT
    pltpu.sync_copy(zeros_hbm.at[pl.ds(base, RPT)], acc.at[pl.ds(base, RPT)])
    pltpu.sync_copy(pk_hbm.at[wid], pk_v)

    def unpack(c, par):
        for t in range(CH // L):
            v = pk_v[c, pl.ds(t * L, L)]
            rbuf[par, pl.ds(t * L, L)] = lax.shift_right_logical(v, SHIFT)
            cbuf[par, pl.ds(t * L, L)] = lax.bitwise_and(v, MASK)

    plsc.subcore_barrier()

    def body(j, carry):
        c = j * 2
        unpack(c, 0)
        unpack(c + 1, 1)
        d0 = pltpu.async_copy(g_hbm.at[rbuf.at[0]], buf0, semg0)
        d1 = pltpu.async_copy(g_hbm.at[rbuf.at[1]], buf1, semg1)
        d0.wait()
        s0 = pltpu.async_copy(buf0, acc.at[cbuf.at[0]], sems0, add=True)
        d1.wait()
        s1 = pltpu.async_copy(buf1, acc.at[cbuf.at[1]], sems1, add=True)
        s0.wait()
        s1.wait()
        return carry

    lax.fori_loop(0, NCHUNK // 2, body, 0)
    plsc.subcore_barrier()
    pltpu.sync_copy(acc.at[pl.ds(base, RPT)],
                    out_hbm.at[cid].at[pl.ds(base, RPT)])


def _prep_body(hist_ref, emb_ref, w_ref, dis_ref, g_ref):
    deg = 1.0 + jnp.sum(hist_ref[...], axis=0)
    dis = lax.rsqrt(deg)[:N, None]
    dis_ref[...] = dis
    h = jnp.dot(emb_ref[...], w_ref[...], preferred_element_type=jnp.float32)
    g_ref[...] = h * dis


def _mid_body(acc_ref, g_ref, dis_ref, b_ref, lnw_ref, lnb_ref, wn_ref,
              gn_ref):
    dis = dis_ref[...]
    s = acc_ref[0, :N, :] + acc_ref[1, :N, :] + g_ref[...]
    y = dis * s + b_ref[...]
    m = jnp.mean(y)
    v = jnp.mean((y - m) ** 2)
    z = (y - m) * lax.rsqrt(v + 1e-5) * lnw_ref[...] + lnb_ref[...]
    z = jnp.maximum(z, 0.0)
    gn_ref[...] = jnp.dot(z, wn_ref[...],
                          preferred_element_type=jnp.float32) * dis


def _fin_body(acc_ref, g_ref, dis_ref, b_ref, lnw_ref, lnb_ref,
              tw1_ref, tb1_ref, tw2_ref, tb2_ref, emb_ref, out_ref):
    s = acc_ref[0, :N, :] + acc_ref[1, :N, :] + g_ref[...]
    y = dis_ref[...] * s + b_ref[...]
    m = jnp.mean(y)
    v = jnp.mean((y - m) ** 2)
    z = (y - m) * lax.rsqrt(v + 1e-5) * lnw_ref[...] + lnb_ref[...]
    t = jnp.dot(z, tw1_ref[...], preferred_element_type=jnp.float32)
    t = jnp.maximum(t + tb1_ref[...], 0.0)
    out_ref[...] = (jnp.dot(t, tw2_ref[...], preferred_element_type=jnp.float32)
                    + tb2_ref[...] + emb_ref[...])


_f32 = jnp.float32
_prep = pl.pallas_call(
    _prep_body,
    out_shape=(jax.ShapeDtypeStruct((N, 1), _f32),
               jax.ShapeDtypeStruct((N, D), _f32)))
_mid = pl.pallas_call(
    _mid_body, out_shape=jax.ShapeDtypeStruct((N, D), _f32))
_fin = pl.pallas_call(
    _fin_body, out_shape=jax.ShapeDtypeStruct((N, D), _f32))


def kernel(edge_index, emb, W0, b0, W1, b1, W2, b2, lnw0, lnb0, lnw1, lnb1,
           lnw2, lnb2, tW1, tb1, tW2, tb2):
    row = edge_index[0]
    col = edge_index[1]
    pad = EPW - EPT
    roww = jnp.concatenate(
        [row.reshape(NW, EPT), jnp.zeros((NW, pad), jnp.int32)], axis=1)
    colw = jnp.concatenate(
        [col.reshape(NW, EPT), jnp.full((NW, pad), N, jnp.int32)], axis=1)
    pk = ((roww << SHIFT) | colw).reshape(NW, NCHUNK, CH)
    zeros = jnp.zeros((NP, D), _f32)

    hist = _hist(col)
    dis, g = _prep(hist, emb, W0)
    layers = [(b0, lnw0, lnb0), (b1, lnw1, lnb1), (b2, lnw2, lnb2)]
    nxt = [W1, W2]
    for i in range(3):
        acc = _scatter(pk, g, zeros)
        b, lw, lb = layers[i]
        if i < 2:
            g = _mid(acc, g, dis, b, lw, lb, nxt[i])
        else:
            return _fin(acc, g, dis, b, lw, lb, tW1, tb1, tW2, tb2, emb)


# speedup vs baseline: 2.3203x; 1.0038x over previous
import functools

import jax
import jax.numpy as jnp
from jax import lax
from jax.experimental import pallas as pl
from jax.experimental.pallas import tpu as pltpu
from jax.experimental.pallas import tpu_sc as plsc

N = 10000
D = 128
E = 320000
NC = 2
NS = 16
L = 16
NW = NC * NS
CH = 128
NCHUNK = 80
EPW = CH * NCHUNK
NP = 10112
RPT = NP // NS
EPT = E // NW
SHIFT = 14
MASK = (1 << SHIFT) - 1

_mesh = plsc.VectorSubcoreMesh(core_axis_name="c", subcore_axis_name="s")


@functools.partial(
    pl.kernel,
    out_type=jax.ShapeDtypeStruct((NW, NP), jnp.float32),
    mesh=_mesh,
    scratch_types=[
        pltpu.VMEM((EPT,), jnp.int32),
        pltpu.VMEM((NP,), jnp.float32),
    ],
    compiler_params=pltpu.CompilerParams(needs_layout_passes=False),
)
def _hist(col_hbm, out_hbm, col_v, hist):
    cid = lax.axis_index("c")
    sid = lax.axis_index("s")
    wid = cid * NS + sid
    pltpu.sync_copy(col_hbm.at[pl.ds(wid * EPT, EPT)], col_v)

    zero16 = jnp.zeros((L,), jnp.float32)

    def zb(i, carry):
        hist[pl.ds(i * L, L)] = zero16
        return carry

    lax.fori_loop(0, NP // L, zb, 0)

    ones16 = jnp.ones((L,), jnp.float32)

    def body(i, carry):
        idx = col_v[pl.ds(i * L, L)]
        plsc.addupdate_scatter(hist, [idx], ones16)
        return carry

    lax.fori_loop(0, EPT // L, body, 0)
    pltpu.sync_copy(hist, out_hbm.at[wid])


@functools.partial(
    pl.kernel,
    out_type=jax.ShapeDtypeStruct((NC, NP, D), jnp.float32),
    mesh=_mesh,
    scratch_types=[
        pltpu.VMEM((NCHUNK, CH), jnp.int32),
        pltpu.VMEM((2, CH), jnp.int32),
        pltpu.VMEM((2, CH), jnp.int32),
        pltpu.VMEM((CH, D), jnp.float32),
        pltpu.VMEM((CH, D), jnp.float32),
        pltpu.VMEM_SHARED((NP, D), jnp.float32),
        pltpu.SemaphoreType.DMA,
        pltpu.SemaphoreType.DMA,
        pltpu.SemaphoreType.DMA,
        pltpu.SemaphoreType.DMA,
    ],
    compiler_params=pltpu.CompilerParams(needs_layout_passes=False),
)
def _scatter(pk_hbm, g_hbm, zeros_hbm, out_hbm,
             pk_v, rbuf, cbuf, buf0, buf1, acc, semg0, semg1, sems0, sems1):
    cid = lax.axis_index("c")
    sid = lax.axis_index("s")
    wid = cid * NS + sid
    base = sid * RPT
    z0 = pltpu.async_copy(zeros_hbm.at[pl.ds(base, RPT)],
                          acc.at[pl.ds(base, RPT)], semg0)
    z1 = pltpu.async_copy(pk_hbm.at[wid], pk_v, semg1)
    z0.wait()
    z1.wait()

    def unpack(c, par):
        for t in range(CH // L):
            v = pk_v[c, pl.ds(t * L, L)]
            rbuf[par, pl.ds(t * L, L)] = lax.shift_right_logical(v, SHIFT)
            cbuf[par, pl.ds(t * L, L)] = lax.bitwise_and(v, MASK)

    plsc.subcore_barrier()

    def body(j, carry):
        c = j * 2
        unpack(c, 0)
        d0 = pltpu.async_copy(g_hbm.at[rbuf.at[0]], buf0, semg0)
        unpack(c + 1, 1)
        d1 = pltpu.async_copy(g_hbm.at[rbuf.at[1]], buf1, semg1)
        d0.wait()
        s0 = pltpu.async_copy(buf0, acc.at[cbuf.at[0]], sems0, add=True)
        d1.wait()
        s1 = pltpu.async_copy(buf1, acc.at[cbuf.at[1]], sems1, add=True)
        s0.wait()
        s1.wait()
        return carry

    lax.fori_loop(0, NCHUNK // 2, body, 0)
    plsc.subcore_barrier()
    pltpu.sync_copy(acc.at[pl.ds(base, RPT)],
                    out_hbm.at[cid].at[pl.ds(base, RPT)])


def _prep_body(hist_ref, emb_ref, w_ref, dis_ref, g_ref):
    deg = 1.0 + jnp.sum(hist_ref[...], axis=0)
    dis = lax.rsqrt(deg)[:N, None]
    dis_ref[...] = dis
    h = jnp.dot(emb_ref[...], w_ref[...], preferred_element_type=jnp.float32)
    g_ref[...] = h * dis


def _mid_body(acc_ref, g_ref, dis_ref, b_ref, lnw_ref, lnb_ref, wn_ref,
              gn_ref):
    dis = dis_ref[...]
    s = acc_ref[0, :N, :] + acc_ref[1, :N, :] + g_ref[...]
    y = dis * s + b_ref[...]
    m = jnp.mean(y)
    v = jnp.mean((y - m) ** 2)
    z = (y - m) * lax.rsqrt(v + 1e-5) * lnw_ref[...] + lnb_ref[...]
    z = jnp.maximum(z, 0.0)
    gn_ref[...] = jnp.dot(z, wn_ref[...],
                          preferred_element_type=jnp.float32) * dis


def _fin_body(acc_ref, g_ref, dis_ref, b_ref, lnw_ref, lnb_ref,
              tw1_ref, tb1_ref, tw2_ref, tb2_ref, emb_ref, out_ref):
    s = acc_ref[0, :N, :] + acc_ref[1, :N, :] + g_ref[...]
    y = dis_ref[...] * s + b_ref[...]
    m = jnp.mean(y)
    v = jnp.mean((y - m) ** 2)
    z = (y - m) * lax.rsqrt(v + 1e-5) * lnw_ref[...] + lnb_ref[...]
    t = jnp.dot(z, tw1_ref[...], preferred_element_type=jnp.float32)
    t = jnp.maximum(t + tb1_ref[...], 0.0)
    out_ref[...] = (jnp.dot(t, tw2_ref[...], preferred_element_type=jnp.float32)
                    + tb2_ref[...] + emb_ref[...])


_f32 = jnp.float32
_prep = pl.pallas_call(
    _prep_body,
    out_shape=(jax.ShapeDtypeStruct((N, 1), _f32),
               jax.ShapeDtypeStruct((N, D), _f32)))
_mid = pl.pallas_call(
    _mid_body, out_shape=jax.ShapeDtypeStruct((N, D), _f32))
_fin = pl.pallas_call(
    _fin_body, out_shape=jax.ShapeDtypeStruct((N, D), _f32))


def kernel(edge_index, emb, W0, b0, W1, b1, W2, b2, lnw0, lnb0, lnw1, lnb1,
           lnw2, lnb2, tW1, tb1, tW2, tb2):
    row = edge_index[0]
    col = edge_index[1]
    pad = EPW - EPT
    roww = jnp.concatenate(
        [row.reshape(NW, EPT), jnp.zeros((NW, pad), jnp.int32)], axis=1)
    colw = jnp.concatenate(
        [col.reshape(NW, EPT), jnp.full((NW, pad), N, jnp.int32)], axis=1)
    pk = ((roww << SHIFT) | colw).reshape(NW, NCHUNK, CH)
    zeros = jnp.zeros((NP, D), _f32)

    hist = _hist(col)
    dis, g = _prep(hist, emb, W0)
    layers = [(b0, lnw0, lnb0), (b1, lnw1, lnb1), (b2, lnw2, lnb2)]
    nxt = [W1, W2]
    for i in range(3):
        acc = _scatter(pk, g, zeros)
        b, lw, lb = layers[i]
        if i < 2:
            g = _mid(acc, g, dis, b, lw, lb, nxt[i])
        else:
            return _fin(acc, g, dis, b, lw, lb, tW1, tb1, tW2, tb2, emb)
